# baseline (speedup 1.0000x reference)
import functools

import jax
import jax.numpy as jnp
from jax import lax
from jax.experimental import pallas as pl
from jax.experimental.pallas import tpu as pltpu
from jax.experimental.pallas import tpu_sc as plsc

_ROWS = 64
_LANES = 16


def _gather_kernel(cam_ref, id_ref, w_ref, b_ref, ws_ref, bs_ref):
    bidx = pl.program_id(0)
    rem = id_ref[bidx] % 8
    sel = jax.lax.broadcasted_iota(jnp.int32, (8, 3), 0) == rem
    ws_ref[0] = jnp.sum(jnp.where(sel, w_ref[0], 0.0), axis=0, keepdims=True)
    bs_ref[0] = jnp.sum(jnp.where(sel, b_ref[0], 0.0), axis=0, keepdims=True)


def _gather_params(camindex, idindex, w, b):
    B = camindex.shape[0]
    return pl.pallas_call(
        _gather_kernel,
        grid_spec=pltpu.PrefetchScalarGridSpec(
            num_scalar_prefetch=2,
            grid=(B,),
            in_specs=[
                pl.BlockSpec((1, 8, 3), lambda bi, cam, idx: (cam[bi], idx[bi] // 8, 0)),
                pl.BlockSpec((1, 8, 3), lambda bi, cam, idx: (cam[bi], idx[bi] // 8, 0)),
            ],
            out_specs=[
                pl.BlockSpec((1, 1, 3), lambda bi, cam, idx: (bi, 0, 0)),
                pl.BlockSpec((1, 1, 3), lambda bi, cam, idx: (bi, 0, 0)),
            ],
        ),
        out_shape=[jax.ShapeDtypeStruct((B, 1, 3), jnp.float32)] * 2,
    )(camindex, idindex, w, b)


def _make_sc_affine(B, C, H, W):
    info = plsc.get_sparse_core_info()
    NC, NS = info.num_cores, info.num_subcores
    NW = NC * NS
    n_planes = B * C
    ppw = -(-n_planes // NW)
    n_chunks = H // _ROWS
    vecs_per_chunk = _ROWS * W // _LANES
    lane_chunks = W // _LANES

    mesh = plsc.VectorSubcoreMesh(core_axis_name="c", subcore_axis_name="s")

    @functools.partial(
        pl.kernel,
        mesh=mesh,
        out_type=jax.ShapeDtypeStruct((B, C, H, W), jnp.float32),
        scratch_types=[
            pltpu.VMEM((n_planes, _LANES), jnp.float32),
            pltpu.VMEM((n_planes, _LANES), jnp.float32),
            pltpu.VMEM((_ROWS, W), jnp.float32),
        ],
    )
    def sc_affine(ws_hbm, bs_hbm, img_hbm, out_hbm, wsv, bsv, buf):
        wid = lax.axis_index("s") * NC + lax.axis_index("c")
        pltpu.sync_copy(ws_hbm, wsv)
        pltpu.sync_copy(bs_hbm, bsv)
        for j in range(ppw):
            p = wid * ppw + j

            @pl.when(p < n_planes)
            def _():
                s = p // C
                c = p % C
                wv = wsv[p, pl.ds(0, _LANES)]
                bv = bsv[p, pl.ds(0, _LANES)]
                for k in range(n_chunks):
                    pltpu.sync_copy(img_hbm.at[s, c, pl.ds(k * _ROWS, _ROWS)], buf)

                    def body(i, carry):
                        r = i // lane_chunks
                        l = (i % lane_chunks) * _LANES
                        buf[r, pl.ds(l, _LANES)] = buf[r, pl.ds(l, _LANES)] * wv + bv
                        return carry

                    lax.fori_loop(0, vecs_per_chunk, body, 0)
                    pltpu.sync_copy(buf, out_hbm.at[s, c, pl.ds(k * _ROWS, _ROWS)])

    return sc_affine


def kernel(image, camindex, idindex, w, b):
    B, C, H, W = image.shape
    ws, bs = _gather_params(camindex, idindex, w, b)
    ws16 = jnp.broadcast_to(ws.reshape(B * C, 1), (B * C, _LANES))
    bs16 = jnp.broadcast_to(bs.reshape(B * C, 1), (B * C, _LANES))
    sc_affine = _make_sc_affine(B, C, H, W)
    return sc_affine(ws16, bs16, image)


# speedup vs baseline: 1.0006x; 1.0006x over previous
import functools

import jax
import jax.numpy as jnp
from jax import lax
from jax.experimental import pallas as pl
from jax.experimental.pallas import tpu as pltpu
from jax.experimental.pallas import tpu_sc as plsc

_ROWS = 64
_LANES = 16


def _gather_kernel(cam_ref, id_ref, w_ref, b_ref, ws_ref, bs_ref):
    bidx = pl.program_id(0)
    rem = id_ref[bidx] % 8
    sel = jax.lax.broadcasted_iota(jnp.int32, (8, 3), 0) == rem
    ws_ref[0] = jnp.sum(jnp.where(sel, w_ref[0], 0.0), axis=0, keepdims=True)
    bs_ref[0] = jnp.sum(jnp.where(sel, b_ref[0], 0.0), axis=0, keepdims=True)


def _gather_params(camindex, idindex, w, b):
    B = camindex.shape[0]
    return pl.pallas_call(
        _gather_kernel,
        grid_spec=pltpu.PrefetchScalarGridSpec(
            num_scalar_prefetch=2,
            grid=(B,),
            in_specs=[
                pl.BlockSpec((1, 8, 3), lambda bi, cam, idx: (cam[bi], idx[bi] // 8, 0)),
                pl.BlockSpec((1, 8, 3), lambda bi, cam, idx: (cam[bi], idx[bi] // 8, 0)),
            ],
            out_specs=[
                pl.BlockSpec((1, 1, 3), lambda bi, cam, idx: (bi, 0, 0)),
                pl.BlockSpec((1, 1, 3), lambda bi, cam, idx: (bi, 0, 0)),
            ],
        ),
        out_shape=[jax.ShapeDtypeStruct((B, 1, 3), jnp.float32)] * 2,
    )(camindex, idindex, w, b)


def _make_sc_affine(B, C, H, W):
    info = plsc.get_sparse_core_info()
    NC, NS = info.num_cores, info.num_subcores
    NW = NC * NS
    n_planes = B * C
    ppw = -(-n_planes // NW)
    n_chunks = H // _ROWS
    vecs_per_chunk = _ROWS * W // _LANES
    lane_chunks = W // _LANES

    mesh = plsc.VectorSubcoreMesh(core_axis_name="c", subcore_axis_name="s")

    @functools.partial(
        pl.kernel,
        mesh=mesh,
        out_type=jax.ShapeDtypeStruct((B, C, H, W), jnp.float32),
        scratch_types=[
            pltpu.VMEM((n_planes, 128), jnp.float32),
            pltpu.VMEM((n_planes, 128), jnp.float32),
            pltpu.VMEM((_ROWS, W), jnp.float32),
        ],
        compiler_params=pltpu.CompilerParams(use_tc_tiling_on_sc=True),
    )
    def sc_affine(ws_hbm, bs_hbm, img_hbm, out_hbm, wsv, bsv, buf):
        wid = lax.axis_index("s") * NC + lax.axis_index("c")
        pltpu.sync_copy(ws_hbm, wsv)
        pltpu.sync_copy(bs_hbm, bsv)
        for j in range(ppw):
            p = wid * ppw + j

            @pl.when(p < n_planes)
            def _():
                s = p // C
                c = p % C
                wv = wsv[p, pl.ds(0, _LANES)]
                bv = bsv[p, pl.ds(0, _LANES)]
                for k in range(n_chunks):
                    pltpu.sync_copy(img_hbm.at[s, c, pl.ds(k * _ROWS, _ROWS)], buf)

                    def body(i, carry):
                        r = i // lane_chunks
                        l = (i % lane_chunks) * _LANES
                        buf[r, pl.ds(l, _LANES)] = buf[r, pl.ds(l, _LANES)] * wv + bv
                        return carry

                    lax.fori_loop(0, vecs_per_chunk, body, 0)
                    pltpu.sync_copy(buf, out_hbm.at[s, c, pl.ds(k * _ROWS, _ROWS)])

    return sc_affine


def kernel(image, camindex, idindex, w, b):
    B, C, H, W = image.shape
    ws, bs = _gather_params(camindex, idindex, w, b)
    ws16 = jnp.broadcast_to(ws.reshape(B * C, 1), (B * C, 128))
    bs16 = jnp.broadcast_to(bs.reshape(B * C, 1), (B * C, 128))
    sc_affine = _make_sc_affine(B, C, H, W)
    return sc_affine(ws16, bs16, image)
